# baseline (speedup 1.0000x reference)
import functools

import jax
import jax.numpy as jnp
from jax import lax
from jax.experimental import pallas as pl
from jax.experimental.pallas import tpu as pltpu
from jax.experimental.pallas import tpu_sc as plsc

_CHUNK = 32


def _build_emb(pe, num_rows):
    vocab, d = pe.shape
    max_rel = (vocab - 1) // 2
    dist = (num_rows - 1) // 2
    info = plsc.get_sparse_core_info()
    nc, lanes = info.num_cores, info.num_lanes
    nw = nc * info.num_subcores
    n_chunks = -(-num_rows // _CHUNK)
    assert n_chunks % nw == 0 and _CHUNK % lanes == 0
    cpw = n_chunks // nw

    mesh = plsc.VectorSubcoreMesh(core_axis_name="c", subcore_axis_name="s")

    @functools.partial(
        pl.kernel,
        mesh=mesh,
        out_type=jax.ShapeDtypeStruct((num_rows, d), jnp.float32),
        scratch_types=[
            pltpu.VMEM((_CHUNK,), jnp.int32),
            pltpu.VMEM((_CHUNK,), jnp.int32),
            pltpu.VMEM((_CHUNK,), jnp.int32),
            pltpu.VMEM((_CHUNK,), jnp.int32),
            pltpu.VMEM((_CHUNK, d), jnp.float32),
            pltpu.VMEM((_CHUNK, d), jnp.float32),
            pltpu.SemaphoreType.DMA,
            pltpu.SemaphoreType.DMA,
            pltpu.SemaphoreType.DMA,
            pltpu.SemaphoreType.DMA,
        ],
    )
    def emb_kernel(
        pe_hbm, out_hbm, gidx0, gidx1, oidx0, oidx1, buf0, buf1, g0, g1, s0, s1
    ):
        gidx_refs, oidx_refs = (gidx0, gidx1), (oidx0, oidx1)
        bufs, gsems, ssems = (buf0, buf1), (g0, g1), (s0, s1)
        wid = lax.axis_index("s") * nc + lax.axis_index("c")

        def start_gather(k):
            b = k % 2
            start = (k * nw + wid) * _CHUNK
            for t in range(_CHUNK // lanes):
                r = start + t * lanes + lax.iota(jnp.int32, lanes)
                gidx_refs[b][pl.ds(t * lanes, lanes)] = (
                    jnp.clip(r - dist, -max_rel, max_rel) + max_rel
                )
                oidx_refs[b][pl.ds(t * lanes, lanes)] = jnp.minimum(r, num_rows - 1)
            cp = pltpu.make_async_copy(pe_hbm.at[gidx_refs[b]], bufs[b], gsems[b])
            cp.start()
            return cp

        def scatter(k):
            b = k % 2
            return pltpu.make_async_copy(bufs[b], out_hbm.at[oidx_refs[b]], ssems[b])

        gathers = {}
        for k in range(cpw):
            if k >= 2:
                scatter(k - 2).wait()
            gathers[k] = start_gather(k)
            if k >= 1:
                gathers[k - 1].wait()
                scatter(k - 1).start()
        gathers[cpw - 1].wait()
        scatter(cpw - 1).start()
        scatter(cpw - 2).wait()
        scatter(cpw - 1).wait()

    return emb_kernel(pe)


def kernel(x, pe):
    return (x, _build_emb(pe, 2 * x.shape[1] - 1))


# speedup vs baseline: 3.0859x; 3.0859x over previous
import functools

import jax
import jax.numpy as jnp
from jax import lax
from jax.experimental import pallas as pl
from jax.experimental.pallas import tpu as pltpu
from jax.experimental.pallas import tpu_sc as plsc

_CHUNK = 32


def _build_emb(pe, num_rows):
    vocab, d = pe.shape
    max_rel = (vocab - 1) // 2
    dist = (num_rows - 1) // 2
    info = plsc.get_sparse_core_info()
    nc, lanes = info.num_cores, info.num_lanes
    nw = nc * info.num_subcores
    n_chunks = -(-num_rows // _CHUNK)
    assert n_chunks % nw == 0 and _CHUNK % lanes == 0
    cpw = n_chunks // nw
    assert cpw >= 2

    mesh = plsc.VectorSubcoreMesh(core_axis_name="c", subcore_axis_name="s")

    @functools.partial(
        pl.kernel,
        mesh=mesh,
        out_type=jax.ShapeDtypeStruct((num_rows, d), jnp.float32),
        scratch_types=[
            pltpu.VMEM((_CHUNK,), jnp.int32),
            pltpu.VMEM((_CHUNK,), jnp.int32),
            pltpu.VMEM((_CHUNK,), jnp.int32),
            pltpu.VMEM((_CHUNK, d), jnp.float32),
            pltpu.VMEM((_CHUNK, d), jnp.float32),
            pltpu.SemaphoreType.DMA,
            pltpu.SemaphoreType.DMA,
            pltpu.SemaphoreType.DMA,
            pltpu.SemaphoreType.DMA,
        ],
    )
    def emb_kernel(
        pe_hbm, out_hbm, gidx0, gidx1, oidx, buf0, buf1, g0, g1, s0, s1
    ):
        gidx_refs, bufs = (gidx0, gidx1), (buf0, buf1)
        gsems, ssems = (g0, g1), (s0, s1)
        wid = lax.axis_index("s") * nc + lax.axis_index("c")
        base = wid * cpw

        def row0(k):
            return (base + k) * _CHUNK

        def gather(k):
            b = k % 2
            for t in range(_CHUNK // lanes):
                r = row0(k) + t * lanes + lax.iota(jnp.int32, lanes)
                gidx_refs[b][pl.ds(t * lanes, lanes)] = (
                    jnp.clip(r - dist, -max_rel, max_rel) + max_rel
                )
            return pltpu.make_async_copy(pe_hbm.at[gidx_refs[b]], bufs[b], gsems[b])

        def uniform_low(k):
            return row0(k) + _CHUNK - 1 <= dist - max_rel

        def uniform_high(k):
            return row0(k) >= dist + max_rel

        def scatter(k):
            b = k % 2
            if k == cpw - 1:
                return pltpu.make_async_copy(bufs[b], out_hbm.at[oidx], ssems[b])
            start = pl.multiple_of(row0(k), _CHUNK)
            return pltpu.make_async_copy(
                bufs[b], out_hbm.at[pl.ds(start, _CHUNK)], ssems[b]
            )

        cp0, cp1 = gather(0), gather(1)
        cp0.start(), cp1.start()
        cp0.wait()
        scatter(0).start()
        cp1.wait()
        scatter(1).start()
        for k in range(2, cpw):
            scatter(k - 2).wait()

            @pl.when(
                jnp.logical_not(
                    (uniform_low(k) & uniform_low(k - 2))
                    | (uniform_high(k) & uniform_high(k - 2))
                )
            )
            def _():
                cp = gather(k)
                cp.start()
                cp.wait()

            if k == cpw - 1:
                for t in range(_CHUNK // lanes):
                    r = row0(k) + t * lanes + lax.iota(jnp.int32, lanes)
                    oidx[pl.ds(t * lanes, lanes)] = jnp.minimum(r, num_rows - 1)
            scatter(k).start()
        scatter(cpw - 2).wait()
        scatter(cpw - 1).wait()

    return emb_kernel(pe)


def kernel(x, pe):
    return (x, _build_emb(pe, 2 * x.shape[1] - 1))


# speedup vs baseline: 3.7248x; 1.2070x over previous
import functools

import jax
import jax.numpy as jnp
from jax import lax
from jax.experimental import pallas as pl
from jax.experimental.pallas import tpu as pltpu
from jax.experimental.pallas import tpu_sc as plsc

_CHUNK = 32


def _build_emb(pe, num_rows):
    vocab, d = pe.shape
    max_rel = (vocab - 1) // 2
    dist = (num_rows - 1) // 2
    info = plsc.get_sparse_core_info()
    nc, lanes = info.num_cores, info.num_lanes
    nw = nc * info.num_subcores
    n_chunks = -(-num_rows // _CHUNK)
    assert n_chunks % nw == 0 and _CHUNK % lanes == 0
    cpw = n_chunks // nw
    assert cpw >= 2

    def _unif_low(c):
        return (c + 1) * _CHUNK - 1 <= dist - max_rel

    def _unif_high(c):
        return c * _CHUNK >= dist + max_rel

    for w in range(nw):
        for k in range(1, cpw - 1):
            c = w * cpw + k
            assert _unif_low(c) or _unif_high(c)

    mesh = plsc.VectorSubcoreMesh(core_axis_name="c", subcore_axis_name="s")

    @functools.partial(
        pl.kernel,
        mesh=mesh,
        out_type=jax.ShapeDtypeStruct((num_rows, d), jnp.float32),
        scratch_types=[
            pltpu.VMEM((_CHUNK,), jnp.int32),
            pltpu.VMEM((_CHUNK,), jnp.int32),
            pltpu.VMEM((_CHUNK,), jnp.int32),
            pltpu.VMEM((_CHUNK, d), jnp.float32),
            pltpu.VMEM((_CHUNK, d), jnp.float32),
            pltpu.SemaphoreType.DMA,
            pltpu.SemaphoreType.DMA,
        ],
    )
    def emb_kernel(pe_hbm, out_hbm, gidx0, gidx1, oidx, buf0, buf1, gsem, ssem):
        wid = lax.axis_index("s") * nc + lax.axis_index("c")
        base = wid * cpw

        def row0(k):
            return (base + k) * _CHUNK

        def fill_gidx(ref, k):
            for t in range(_CHUNK // lanes):
                r = row0(k) + t * lanes + lax.iota(jnp.int32, lanes)
                ref[pl.ds(t * lanes, lanes)] = (
                    jnp.clip(r - dist, -max_rel, max_rel) + max_rel
                )

        def uniform_low(k):
            return row0(k) + _CHUNK - 1 <= dist - max_rel

        def uniform_high(k):
            return row0(k) >= dist + max_rel

        def linear_dst(k):
            return out_hbm.at[pl.ds(pl.multiple_of(row0(k), _CHUNK), _CHUNK)]

        fill_gidx(gidx0, 0)
        fill_gidx(gidx1, cpw - 1)
        for t in range(_CHUNK // lanes):
            r = row0(cpw - 1) + t * lanes + lax.iota(jnp.int32, lanes)
            oidx[pl.ds(t * lanes, lanes)] = jnp.minimum(r, num_rows - 1)
        cp0 = pltpu.make_async_copy(pe_hbm.at[gidx0], buf0, gsem)
        cp1 = pltpu.make_async_copy(pe_hbm.at[gidx1], buf1, gsem)
        cp0.start()
        cp1.start()
        cp0.wait()
        cp1.wait()

        pltpu.make_async_copy(buf0, linear_dst(0), ssem).start()
        for k in range(1, cpw - 1):
            from_first = (uniform_low(k) & uniform_low(0)) | (
                uniform_high(k) & uniform_high(0)
            )

            @pl.when(from_first)
            def _(k=k):
                pltpu.make_async_copy(buf0, linear_dst(k), ssem).start()

            @pl.when(jnp.logical_not(from_first))
            def _(k=k):
                pltpu.make_async_copy(buf1, linear_dst(k), ssem).start()

        pltpu.make_async_copy(buf1, out_hbm.at[oidx], ssem).start()

        for k in range(cpw - 1):
            pltpu.make_async_copy(buf0, linear_dst(k), ssem).wait()
        pltpu.make_async_copy(buf1, out_hbm.at[oidx], ssem).wait()

    return emb_kernel(pe)


def kernel(x, pe):
    return (x, _build_emb(pe, 2 * x.shape[1] - 1))
